# baseline (speedup 1.0000x reference)
import functools

import jax
import jax.numpy as jnp
from jax import lax
from jax.experimental import pallas as pl
from jax.experimental.pallas import tpu as pltpu
from jax.experimental.pallas import tpu_sc as plsc

NC = 2
NS = 16
NW = NC * NS
NCH = 33
CW = 96
NBUF = 4
L = 16
N = 100000
D = 256
V = 41
STRIDE = NCH * CW
LAST = N - STRIDE


def _sc_gather(table, idx):
    mesh = plsc.VectorSubcoreMesh(core_axis_name="c", subcore_axis_name="s")

    @functools.partial(
        pl.kernel,
        out_type=jax.ShapeDtypeStruct((N, D), table.dtype),
        mesh=mesh,
        scratch_types=[
            pltpu.VMEM((STRIDE,), jnp.int32),
            pltpu.VMEM((V, D), jnp.float32),
            pltpu.VMEM((NBUF * CW, D), jnp.float32),
            [pltpu.SemaphoreType.DMA] * NBUF,
        ],
        compiler_params=pltpu.CompilerParams(needs_layout_passes=False),
    )
    def run(table_hbm, idx_hbm, out_hbm, idx_v, tab_v, buf_v, sems):
        wid = lax.axis_index("s") * NC + lax.axis_index("c")
        base = lax.min(wid * STRIDE, LAST)
        pltpu.sync_copy(table_hbm, tab_v)
        pltpu.sync_copy(idx_hbm.at[pl.ds(base, STRIDE)], idx_v)

        io = lax.broadcasted_iota(jnp.int32, (L,), 0)
        cols = [io + L * c for c in range(D // L)]
        lanes = [jnp.full((L,), l, jnp.int32) for l in range(L)]

        def splat(r_vec, l):
            return lax.gather(
                r_vec, lanes[l][:, None],
                dimension_numbers=lax.GatherDimensionNumbers(
                    offset_dims=(), collapsed_slice_dims=(0,),
                    start_index_map=(0,)),
                slice_sizes=(1,),
                mode=lax.GatherScatterMode.PROMISE_IN_BOUNDS)

        def build(g, buf):
            def group(i, carry):
                r_vec = idx_v[pl.ds(g * CW + i * L, L)]
                row0 = buf * CW + i * L

                def loads(l):
                    rs = splat(r_vec, l)
                    return [plsc.load_gather(tab_v, [rs, cols[c]])
                            for c in range(D // L)]

                def stores(l, vals):
                    for c in range(D // L):
                        buf_v[row0 + l, pl.ds(L * c, L)] = vals[c]

                def load_store(l, vals):
                    rs = splat(r_vec, l)
                    cur = []
                    for c in range(D // L):
                        cur.append(plsc.load_gather(tab_v, [rs, cols[c]]))
                        buf_v[row0 + l - 1, pl.ds(L * c, L)] = vals[c]
                    return cur

                prev = loads(0)
                for l in range(1, L):
                    prev = load_store(l, prev)
                stores(L - 1, prev)
                return carry
            lax.fori_loop(0, CW // L, group, 0)

        def scatter(g, buf):
            pltpu.async_copy(
                buf_v.at[pl.ds(buf * CW, CW)],
                out_hbm.at[pl.ds(base + g * CW, CW)], sems[buf])

        def wait(buf):
            pltpu.make_async_copy(
                buf_v.at[pl.ds(buf * CW, CW)],
                out_hbm.at[pl.ds(0, CW)], sems[buf]).wait()

        def step(g, carry):
            b = g % NBUF
            for k in range(NBUF):
                @pl.when((g >= NBUF) & (b == k))
                def _():
                    wait(k)
            build(g, b)
            for k in range(NBUF):
                @pl.when(b == k)
                def _():
                    scatter(g, k)
            return carry

        lax.fori_loop(0, NCH, step, 0)
        for k in range(NBUF):
            wait(k)

    return run(table, idx)


def kernel(emb0, x):
    return _sc_gather(emb0, x.astype(jnp.int32).reshape(N))


# speedup vs baseline: 1.0141x; 1.0141x over previous
import functools

import jax
import jax.numpy as jnp
from jax import lax
from jax.experimental import pallas as pl
from jax.experimental.pallas import tpu as pltpu
from jax.experimental.pallas import tpu_sc as plsc

NC = 2
NS = 16
NW = NC * NS
NCH = 18
CW = 176
L = 16
N = 100000
D = 256
V = 41
STRIDE = NCH * CW
LAST = N - STRIDE


def _sc_gather(table, idx):
    mesh = plsc.VectorSubcoreMesh(core_axis_name="c", subcore_axis_name="s")

    @functools.partial(
        pl.kernel,
        out_type=jax.ShapeDtypeStruct((N, D), table.dtype),
        mesh=mesh,
        scratch_types=[
            pltpu.VMEM((STRIDE,), jnp.int32),
            pltpu.VMEM((V, D), jnp.float32),
            pltpu.VMEM((2 * CW, D), jnp.float32),
            pltpu.SemaphoreType.DMA,
            pltpu.SemaphoreType.DMA,
        ],
        compiler_params=pltpu.CompilerParams(needs_layout_passes=False),
    )
    def run(table_hbm, idx_hbm, out_hbm, idx_v, tab_v, buf_v, sem0, sem1):
        wid = lax.axis_index("s") * NC + lax.axis_index("c")
        base = lax.min(wid * STRIDE, LAST)
        c_tab = pltpu.async_copy(table_hbm, tab_v, sem0)
        c_idx = pltpu.async_copy(idx_hbm.at[pl.ds(base, STRIDE)], idx_v, sem1)
        c_tab.wait()
        c_idx.wait()

        io = lax.broadcasted_iota(jnp.int32, (L,), 0)
        cols = [io + L * c for c in range(D // L)]
        lanes = [jnp.full((L,), l, jnp.int32) for l in range(L)]

        def splat(r_vec, l):
            return lax.gather(
                r_vec, lanes[l][:, None],
                dimension_numbers=lax.GatherDimensionNumbers(
                    offset_dims=(), collapsed_slice_dims=(0,),
                    start_index_map=(0,)),
                slice_sizes=(1,),
                mode=lax.GatherScatterMode.PROMISE_IN_BOUNDS)

        def build(g, half):
            def group(i, carry):
                r_vec = idx_v[pl.ds(g * CW + i * L, L)]
                row0 = half * CW + i * L

                def loads(l):
                    rs = splat(r_vec, l)
                    return [plsc.load_gather(tab_v, [rs, cols[c]])
                            for c in range(D // L)]

                def stores(l, vals):
                    for c in range(D // L):
                        buf_v[row0 + l, pl.ds(L * c, L)] = vals[c]

                def load_store(l, vals):
                    rs = splat(r_vec, l)
                    cur = []
                    for c in range(D // L):
                        cur.append(plsc.load_gather(tab_v, [rs, cols[c]]))
                        buf_v[row0 + l - 1, pl.ds(L * c, L)] = vals[c]
                    return cur

                prev = loads(0)
                for l in range(1, L):
                    prev = load_store(l, prev)
                stores(L - 1, prev)
                return carry
            lax.fori_loop(0, CW // L, group, 0)

        def scatter(g, half, sem):
            return pltpu.async_copy(
                buf_v.at[pl.ds(half * CW, CW)],
                out_hbm.at[pl.ds(base + g * CW, CW)], sem)

        def wait(half, sem):
            pltpu.make_async_copy(
                buf_v.at[pl.ds(half * CW, CW)],
                out_hbm.at[pl.ds(0, CW)], sem).wait()

        def step(g, carry):
            half = g % 2

            @pl.when((g >= 2) & (half == 0))
            def _():
                wait(0, sem0)

            @pl.when((g >= 2) & (half == 1))
            def _():
                wait(1, sem1)

            build(g, half)

            @pl.when(half == 0)
            def _():
                scatter(g, 0, sem0)

            @pl.when(half == 1)
            def _():
                scatter(g, 1, sem1)

            return carry

        lax.fori_loop(0, NCH, step, 0)
        wait(0, sem0)
        wait(1, sem1)

    return run(table, idx)


def kernel(emb0, x):
    return _sc_gather(emb0, x.astype(jnp.int32).reshape(N))
